# Initial kernel scaffold; baseline (speedup 1.0000x reference)
import jax
import jax.numpy as jnp
from jax.experimental import pallas as pl


def kernel(x, table):
    raise NotImplementedError("write your pallas kernel here")


# speedup vs baseline: 1.5594x; 1.5594x over previous
import jax
import jax.numpy as jnp
from jax import lax
from jax.experimental import pallas as pl
from jax.experimental.pallas import tpu as pltpu
from jax.experimental.pallas import tpu_sc as plsc

BATCH = 16384
N_FIELDS = 26
OUT_DIM = 32
TOTAL = BATCH * N_FIELDS
NC = 2
NS = 16
NW = NC * NS
PER_W = TOTAL // NW
STREAM = 128
K = 8
GROUP_ROWS = STREAM * K
NGROUPS = PER_W // GROUP_ROWS


def _gather_body(table_hbm, idx_hbm, out_hbm, idx_v, rows_v, sem):
    wid = lax.axis_index("s") * NC + lax.axis_index("c")
    base = wid * PER_W
    pltpu.sync_copy(idx_hbm.at[pl.ds(base, PER_W)], idx_v)

    def group(g, carry):
        goff = g * GROUP_ROWS
        copies = []
        for b in range(K):
            idx_sl = idx_v.at[pl.ds(goff + b * STREAM, STREAM)]
            copies.append(
                pltpu.async_copy(
                    table_hbm.at[idx_sl],
                    rows_v.at[pl.ds(b * STREAM, STREAM), :],
                    sem,
                )
            )
        for c in copies:
            c.wait()
        pltpu.sync_copy(rows_v, out_hbm.at[pl.ds(base + goff, GROUP_ROWS), :])
        return carry

    lax.fori_loop(0, NGROUPS, group, 0)


def kernel(x, table):
    idx = x.reshape(-1).astype(jnp.int32)
    mesh = plsc.VectorSubcoreMesh(core_axis_name="c", subcore_axis_name="s")
    f = pl.kernel(
        _gather_body,
        mesh=mesh,
        out_type=jax.ShapeDtypeStruct((TOTAL, OUT_DIM), jnp.float32),
        scratch_types=[
            pltpu.VMEM((PER_W,), jnp.int32),
            pltpu.VMEM((GROUP_ROWS, OUT_DIM), jnp.float32),
            pltpu.SemaphoreType.DMA,
        ],
        compiler_params=pltpu.CompilerParams(use_tc_tiling_on_sc=False),
    )
    out = f(table, idx)
    return out.reshape(BATCH, N_FIELDS, OUT_DIM)


# speedup vs baseline: 1.5762x; 1.0107x over previous
import jax
import jax.numpy as jnp
from jax import lax
from jax.experimental import pallas as pl
from jax.experimental.pallas import tpu as pltpu
from jax.experimental.pallas import tpu_sc as plsc

BATCH = 16384
N_FIELDS = 26
OUT_DIM = 32
TOTAL = BATCH * N_FIELDS
NC = 2
NS = 16
NW = NC * NS
PER_W = TOTAL // NW
STREAM = 128
K = 13
GROUP_ROWS = STREAM * K
NGROUPS = PER_W // GROUP_ROWS


def _gather_body(table_hbm, idx_hbm, out_hbm,
                 idx_v, buf0, buf1, gs0, gs1, ws0, ws1):
    wid = lax.axis_index("s") * NC + lax.axis_index("c")
    base = wid * PER_W
    pltpu.sync_copy(idx_hbm.at[pl.ds(base, PER_W)], idx_v)

    bufs = (buf0, buf1)
    gsems = (gs0, gs1)
    wsems = (ws0, ws1)

    def fire(g, p):
        for b in range(K):
            idx_sl = idx_v.at[pl.ds(g * GROUP_ROWS + b * STREAM, STREAM)]
            pltpu.async_copy(
                table_hbm.at[idx_sl],
                bufs[p].at[pl.ds(b * STREAM, STREAM), :],
                gsems[p],
            )

    def wait_gathers(p):
        pltpu.make_async_copy(
            table_hbm.at[pl.ds(0, GROUP_ROWS)], bufs[p], gsems[p]
        ).wait()

    def start_wb(g, p):
        pltpu.async_copy(
            bufs[p], out_hbm.at[pl.ds(base + g * GROUP_ROWS, GROUP_ROWS), :],
            wsems[p],
        )

    def wait_wb(p):
        pltpu.make_async_copy(
            bufs[p], out_hbm.at[pl.ds(base, GROUP_ROWS), :], wsems[p]
        ).wait()

    fire(0, 0)
    fire(1, 1)

    def pair(j, carry):
        for p in (0, 1):
            g = 2 * j + p
            wait_gathers(p)
            start_wb(g, p)
            wait_wb(p)
            fire(g + 2, p)
        return carry

    lax.fori_loop(0, NGROUPS // 2 - 1, pair, 0)

    for p in (0, 1):
        wait_gathers(p)
        start_wb(NGROUPS - 2 + p, p)
    for p in (0, 1):
        wait_wb(p)


def kernel(x, table):
    idx = x.reshape(-1).astype(jnp.int32)
    mesh = plsc.VectorSubcoreMesh(core_axis_name="c", subcore_axis_name="s")
    f = pl.kernel(
        _gather_body,
        mesh=mesh,
        out_type=jax.ShapeDtypeStruct((TOTAL, OUT_DIM), jnp.float32),
        scratch_types=[
            pltpu.VMEM((PER_W,), jnp.int32),
            pltpu.VMEM((GROUP_ROWS, OUT_DIM), jnp.float32),
            pltpu.VMEM((GROUP_ROWS, OUT_DIM), jnp.float32),
            pltpu.SemaphoreType.DMA,
            pltpu.SemaphoreType.DMA,
            pltpu.SemaphoreType.DMA,
            pltpu.SemaphoreType.DMA,
        ],
        compiler_params=pltpu.CompilerParams(use_tc_tiling_on_sc=False),
    )
    out = f(table, idx)
    return out.reshape(BATCH, N_FIELDS, OUT_DIM)
